# Initial kernel scaffold; baseline (speedup 1.0000x reference)
import jax
import jax.numpy as jnp
from jax.experimental import pallas as pl


def kernel(hidden_state, W1, b1, W2, b2):
    raise NotImplementedError("write your pallas kernel here")


# speedup vs baseline: 45.6659x; 45.6659x over previous
import functools

import jax
import jax.numpy as jnp
import numpy as np
from jax.experimental import pallas as pl
from jax.experimental.pallas import tpu as pltpu

_SPARSITY = 0.9
_INT_MIN = np.int32(-2147483648)
_INT_MAXPOS = np.int32(0x7FFFFFFF)


def _to_float(u):
    bits = u ^ (jnp.right_shift(u, 31) & _INT_MAXPOS)
    return jax.lax.bitcast_convert_type(bits, jnp.float32)


def _router_kernel(x_ref, w1_ref, b1_ref, w2_ref, b2_ref, scores_ref,
                   mask_ref, *, k):
    h = jnp.dot(x_ref[...], w1_ref[...],
                preferred_element_type=jnp.float32)
    h = h + b1_ref[...]
    h = 0.5 * h * (1.0 + jax.lax.erf(h * np.float32(0.7071067811865476)))
    s = jnp.dot(h, w2_ref[...],
                preferred_element_type=jnp.float32)
    s = s + b2_ref[...]
    scores_ref[...] = s

    rows = s.shape[0]

    def body(i, prefix):
        bit = jnp.left_shift(np.int32(1), np.int32(31) - i)
        cand = prefix | bit
        thr = _to_float(cand ^ _INT_MIN)
        cnt = jnp.sum((scores_ref[...] >= thr).astype(jnp.int32), axis=1,
                      keepdims=True)
        return jnp.where(cnt >= k, cand, prefix)

    prefix = jax.lax.fori_loop(
        0, 32, body, jnp.zeros((rows, 1), jnp.int32), unroll=True)
    thr = _to_float(prefix ^ _INT_MIN)
    mask_ref[...] = (scores_ref[...] >= thr).astype(jnp.float32)


@functools.partial(jax.jit, static_argnames=("block_rows",))
def _router(x, W1, b1, W2, b2, block_rows=64):
    n, hidden = x.shape
    ffn = W2.shape[1]
    k = max(1, int(ffn * (1.0 - _SPARSITY)))
    grid = (n // block_rows,)
    scores, mask = pl.pallas_call(
        functools.partial(_router_kernel, k=k),
        grid=grid,
        in_specs=[
            pl.BlockSpec((block_rows, hidden), lambda i: (i, 0)),
            pl.BlockSpec((hidden, W1.shape[1]), lambda i: (0, 0)),
            pl.BlockSpec((1, W1.shape[1]), lambda i: (0, 0)),
            pl.BlockSpec((W1.shape[1], ffn), lambda i: (0, 0)),
            pl.BlockSpec((1, ffn), lambda i: (0, 0)),
        ],
        out_specs=[
            pl.BlockSpec((block_rows, ffn), lambda i: (i, 0)),
            pl.BlockSpec((block_rows, ffn), lambda i: (i, 0)),
        ],
        out_shape=[
            jax.ShapeDtypeStruct((n, ffn), jnp.float32),
            jax.ShapeDtypeStruct((n, ffn), jnp.float32),
        ],
        compiler_params=pltpu.CompilerParams(
            dimension_semantics=("arbitrary",),
        ),
    )(x, W1, b1.reshape(1, -1), W2, b2.reshape(1, -1))
    return scores, mask


def kernel(hidden_state, W1, b1, W2, b2):
    b, s, hidden = hidden_state.shape
    x = hidden_state.reshape(b * s, hidden)
    scores, mask = _router(x, W1, b1, W2, b2)
    ffn = W2.shape[1]
    return scores.reshape(b, s, ffn), mask.reshape(b, s, ffn)


# speedup vs baseline: 50.6116x; 1.1083x over previous
import functools

import jax
import jax.numpy as jnp
import numpy as np
from jax.experimental import pallas as pl
from jax.experimental.pallas import tpu as pltpu

_SPARSITY = 0.9
_INT_MIN = np.int32(-2147483648)
_INT_MAXPOS = np.int32(0x7FFFFFFF)


def _to_float(u):
    bits = u ^ (jnp.right_shift(u, 31) & _INT_MAXPOS)
    return jax.lax.bitcast_convert_type(bits, jnp.float32)


def _router_kernel(x_ref, w1_ref, b1_ref, w2_ref, b2_ref, scores_ref,
                   mask_ref, *, k):
    h = jnp.dot(x_ref[...], w1_ref[...],
                preferred_element_type=jnp.float32)
    h = h + b1_ref[...]
    h = 0.5 * h * (1.0 + jax.lax.erf(h * np.float32(0.7071067811865476)))
    s = jnp.dot(h, w2_ref[...],
                preferred_element_type=jnp.float32)
    s = s + b2_ref[...]
    scores_ref[...] = s

    rows = s.shape[0]

    def body(i, prefix):
        bit = jnp.left_shift(np.int32(1), np.int32(31) - i)
        cand = prefix | bit
        thr = _to_float(cand ^ _INT_MIN)
        cnt = jnp.sum((scores_ref[...] >= thr).astype(jnp.int32), axis=1,
                      keepdims=True)
        return jnp.where(cnt >= k, cand, prefix)

    prefix = jax.lax.fori_loop(
        0, 32, body, jnp.zeros((rows, 1), jnp.int32), unroll=True)
    thr = _to_float(prefix ^ _INT_MIN)
    mask_ref[...] = (scores_ref[...] >= thr).astype(jnp.float32)


@functools.partial(jax.jit, static_argnames=("block_rows",))
def _router(x, W1, b1, W2, b2, block_rows=256):
    n, hidden = x.shape
    ffn = W2.shape[1]
    k = max(1, int(ffn * (1.0 - _SPARSITY)))
    grid = (n // block_rows,)
    scores, mask = pl.pallas_call(
        functools.partial(_router_kernel, k=k),
        grid=grid,
        in_specs=[
            pl.BlockSpec((block_rows, hidden), lambda i: (i, 0)),
            pl.BlockSpec((hidden, W1.shape[1]), lambda i: (0, 0)),
            pl.BlockSpec((1, W1.shape[1]), lambda i: (0, 0)),
            pl.BlockSpec((W1.shape[1], ffn), lambda i: (0, 0)),
            pl.BlockSpec((1, ffn), lambda i: (0, 0)),
        ],
        out_specs=[
            pl.BlockSpec((block_rows, ffn), lambda i: (i, 0)),
            pl.BlockSpec((block_rows, ffn), lambda i: (i, 0)),
        ],
        out_shape=[
            jax.ShapeDtypeStruct((n, ffn), jnp.float32),
            jax.ShapeDtypeStruct((n, ffn), jnp.float32),
        ],
        compiler_params=pltpu.CompilerParams(
            dimension_semantics=("arbitrary",),
        ),
    )(x, W1, b1.reshape(1, -1), W2, b2.reshape(1, -1))
    return scores, mask


def kernel(hidden_state, W1, b1, W2, b2):
    b, s, hidden = hidden_state.shape
    x = hidden_state.reshape(b * s, hidden)
    scores, mask = _router(x, W1, b1, W2, b2)
    ffn = W2.shape[1]
    return scores.reshape(b, s, ffn), mask.reshape(b, s, ffn)


# speedup vs baseline: 68.9828x; 1.3630x over previous
import functools

import jax
import jax.numpy as jnp
import numpy as np
from jax.experimental import pallas as pl
from jax.experimental.pallas import tpu as pltpu

_SPARSITY = 0.9
_INT_MIN = np.int32(-2147483648)
_INT_MAXPOS = np.int32(0x7FFFFFFF)


def _to_float(u):
    bits = u ^ (jnp.right_shift(u, 31) & _INT_MAXPOS)
    return jax.lax.bitcast_convert_type(bits, jnp.float32)


def _router_kernel(x_ref, w1_ref, b1_ref, w2_ref, b2_ref, scores_ref,
                   mask_ref, *, k):
    h = jnp.dot(x_ref[...], w1_ref[...],
                preferred_element_type=jnp.float32)
    h = h + b1_ref[...]
    h = 0.5 * h * (1.0 + jax.lax.erf(h * np.float32(0.7071067811865476)))
    s = jnp.dot(h, w2_ref[...],
                preferred_element_type=jnp.float32)
    s = s + b2_ref[...]
    scores_ref[...] = s

    rows = s.shape[0]

    bits = jax.lax.bitcast_convert_type(s, jnp.int32)
    u = bits ^ (jnp.right_shift(bits, 31) & _INT_MAXPOS)
    hi = jnp.right_shift(u, 16)
    k1 = hi.astype(jnp.int16)

    def _count_ge(keys, thr):
        m = jnp.where(keys >= thr, np.int16(1), np.int16(0))
        cols = keys.shape[1]
        c1 = min(2048, cols)
        part = m[:, :c1]
        for off in range(c1, cols, c1):
            part = part + m[:, off:off + c1]
        c2 = min(256, c1)
        part2 = part[:, :c2]
        for off in range(c2, c1, c2):
            part2 = part2 + part[:, off:off + c2]
        return jnp.sum(part2.astype(jnp.int32), axis=1, keepdims=True)

    def body1(i, prefix):
        cand = prefix | jnp.left_shift(np.int32(1), np.int32(15) - i)
        thr = (cand - np.int32(32768)).astype(jnp.int16)
        cnt = _count_ge(k1, thr)
        return jnp.where(cnt >= k, cand, prefix)

    p1 = jax.lax.fori_loop(0, 16, body1, jnp.zeros((rows, 1), jnp.int32),
                           unroll=True)
    t_hi = p1 - np.int32(32768)

    thr_next = jnp.minimum(p1 - np.int32(32767),
                           np.int32(32767)).astype(jnp.int16)
    cnt_gt = jnp.where(p1 >= np.int32(65535), np.int32(0),
                       _count_ge(k1, thr_next))
    r = k - cnt_gt

    low = (u & np.int32(0xFFFF)) - np.int32(32768)
    k2 = jnp.where(hi == t_hi, low, np.int32(-32768)).astype(jnp.int16)

    def body2(i, prefix):
        cand = prefix | jnp.left_shift(np.int32(1), np.int32(15) - i)
        thr = (cand - np.int32(32768)).astype(jnp.int16)
        cnt = _count_ge(k2, thr)
        return jnp.where(cnt >= r, cand, prefix)

    p2 = jax.lax.fori_loop(0, 16, body2, jnp.zeros((rows, 1), jnp.int32),
                           unroll=True)

    u_thr = jnp.left_shift(t_hi, 16) | p2
    thr = _to_float(u_thr)
    mask_ref[...] = (s >= thr).astype(jnp.float32)


@functools.partial(jax.jit, static_argnames=("block_rows",))
def _router(x, W1, b1, W2, b2, block_rows=128):
    n, hidden = x.shape
    ffn = W2.shape[1]
    k = max(1, int(ffn * (1.0 - _SPARSITY)))
    grid = (n // block_rows,)
    scores, mask = pl.pallas_call(
        functools.partial(_router_kernel, k=k),
        grid=grid,
        in_specs=[
            pl.BlockSpec((block_rows, hidden), lambda i: (i, 0)),
            pl.BlockSpec((hidden, W1.shape[1]), lambda i: (0, 0)),
            pl.BlockSpec((1, W1.shape[1]), lambda i: (0, 0)),
            pl.BlockSpec((W1.shape[1], ffn), lambda i: (0, 0)),
            pl.BlockSpec((1, ffn), lambda i: (0, 0)),
        ],
        out_specs=[
            pl.BlockSpec((block_rows, ffn), lambda i: (i, 0)),
            pl.BlockSpec((block_rows, ffn), lambda i: (i, 0)),
        ],
        out_shape=[
            jax.ShapeDtypeStruct((n, ffn), jnp.float32),
            jax.ShapeDtypeStruct((n, ffn), jnp.float32),
        ],
        compiler_params=pltpu.CompilerParams(
            dimension_semantics=("arbitrary",),
        ),
    )(x, W1, b1.reshape(1, -1), W2, b2.reshape(1, -1))
    return scores, mask


def kernel(hidden_state, W1, b1, W2, b2):
    b, s, hidden = hidden_state.shape
    x = hidden_state.reshape(b * s, hidden)
    scores, mask = _router(x, W1, b1, W2, b2)
    ffn = W2.shape[1]
    return scores.reshape(b, s, ffn), mask.reshape(b, s, ffn)


# speedup vs baseline: 72.2996x; 1.0481x over previous
import functools

import jax
import jax.numpy as jnp
import numpy as np
from jax.experimental import pallas as pl
from jax.experimental.pallas import tpu as pltpu

_SPARSITY = 0.9
_INT_MIN = np.int32(-2147483648)
_INT_MAXPOS = np.int32(0x7FFFFFFF)


def _to_float(u):
    bits = u ^ (jnp.right_shift(u, 31) & _INT_MAXPOS)
    return jax.lax.bitcast_convert_type(bits, jnp.float32)


def _router_kernel(x_ref, w1_ref, b1_ref, w2_ref, b2_ref, scores_ref,
                   mask_ref, *, k):
    h = jnp.dot(x_ref[...], w1_ref[...],
                preferred_element_type=jnp.float32)
    h = h + b1_ref[...]
    h = 0.5 * h * (1.0 + jax.lax.erf(h * np.float32(0.7071067811865476)))
    s = jnp.dot(h, w2_ref[...],
                preferred_element_type=jnp.float32)
    s = s + b2_ref[...]
    scores_ref[...] = s

    rows = s.shape[0]

    bits = jax.lax.bitcast_convert_type(s, jnp.int32)
    t16 = jnp.right_shift(bits, 16)
    k1 = (t16 ^ (jnp.right_shift(t16, 15) & np.int32(0x7FFF))).astype(jnp.int16)

    def _count_ge(keys, thr):
        m = jnp.where(keys >= thr, np.int16(1), np.int16(0))
        cols = keys.shape[1]
        c1 = min(2048, cols)
        part = m[:, :c1]
        for off in range(c1, cols, c1):
            part = part + m[:, off:off + c1]
        c2 = min(256, c1)
        part2 = part[:, :c2]
        for off in range(c2, c1, c2):
            part2 = part2 + part[:, off:off + c2]
        return jnp.sum(part2.astype(jnp.int32), axis=1, keepdims=True)

    def body1(i, prefix):
        cand = prefix | jnp.left_shift(np.int32(1), np.int32(15) - i)
        thr = (cand - np.int32(32768)).astype(jnp.int16)
        cnt = _count_ge(k1, thr)
        return jnp.where(cnt >= k, cand, prefix)

    p1 = jax.lax.fori_loop(0, 16, body1, jnp.zeros((rows, 1), jnp.int32),
                           unroll=True)
    t_hi = p1 - np.int32(32768)

    thr_next = jnp.minimum(p1 - np.int32(32767),
                           np.int32(32767)).astype(jnp.int16)
    cnt_gt = jnp.where(p1 >= np.int32(65535), np.int32(0),
                       _count_ge(k1, thr_next))
    r = k - cnt_gt

    low = ((bits ^ jnp.right_shift(bits, 31)) & np.int32(0xFFFF)) \
        - np.int32(32768)
    k2 = jnp.where(k1 == t_hi.astype(jnp.int16), low.astype(jnp.int16),
                   np.int16(-32768))

    def body2(i, prefix):
        cand = prefix | jnp.left_shift(np.int32(1), np.int32(15) - i)
        thr = (cand - np.int32(32768)).astype(jnp.int16)
        cnt = _count_ge(k2, thr)
        return jnp.where(cnt >= r, cand, prefix)

    p2 = jax.lax.fori_loop(0, 16, body2, jnp.zeros((rows, 1), jnp.int32),
                           unroll=True)

    u_thr = jnp.left_shift(t_hi, 16) | p2
    thr = _to_float(u_thr)
    mask_ref[...] = (s >= thr).astype(jnp.float32)


@functools.partial(jax.jit, static_argnames=("block_rows",))
def _router(x, W1, b1, W2, b2, block_rows=256):
    n, hidden = x.shape
    ffn = W2.shape[1]
    k = max(1, int(ffn * (1.0 - _SPARSITY)))
    grid = (n // block_rows,)
    scores, mask = pl.pallas_call(
        functools.partial(_router_kernel, k=k),
        grid=grid,
        in_specs=[
            pl.BlockSpec((block_rows, hidden), lambda i: (i, 0)),
            pl.BlockSpec((hidden, W1.shape[1]), lambda i: (0, 0)),
            pl.BlockSpec((1, W1.shape[1]), lambda i: (0, 0)),
            pl.BlockSpec((W1.shape[1], ffn), lambda i: (0, 0)),
            pl.BlockSpec((1, ffn), lambda i: (0, 0)),
        ],
        out_specs=[
            pl.BlockSpec((block_rows, ffn), lambda i: (i, 0)),
            pl.BlockSpec((block_rows, ffn), lambda i: (i, 0)),
        ],
        out_shape=[
            jax.ShapeDtypeStruct((n, ffn), jnp.float32),
            jax.ShapeDtypeStruct((n, ffn), jnp.float32),
        ],
        compiler_params=pltpu.CompilerParams(
            dimension_semantics=("arbitrary",),
        ),
    )(x, W1, b1.reshape(1, -1), W2, b2.reshape(1, -1))
    return scores, mask


def kernel(hidden_state, W1, b1, W2, b2):
    b, s, hidden = hidden_state.shape
    x = hidden_state.reshape(b * s, hidden)
    scores, mask = _router(x, W1, b1, W2, b2)
    ffn = W2.shape[1]
    return scores.reshape(b, s, ffn), mask.reshape(b, s, ffn)
